# Initial kernel scaffold; baseline (speedup 1.0000x reference)
import jax
import jax.numpy as jnp
from jax.experimental import pallas as pl


def kernel(V, E, edge_index, ew1, eb1, ew2, eb2, vw1, vb1, vw2, vb2, ow1, ob1, ow2, ob2):
    raise NotImplementedError("write your pallas kernel here")


# speedup vs baseline: 3.5993x; 3.5993x over previous
import functools

import jax
import jax.numpy as jnp
from jax import lax
from jax.experimental import pallas as pl
from jax.experimental.pallas import tpu as pltpu
from jax.experimental.pallas import tpu_sc as plsc

N_NODES = 50000
N_EDGES = 1600000
NP = 51200
D = 8
NC = 2
NS = 16
NW = NC * NS
CH = 2000

_sc_mesh = plsc.VectorSubcoreMesh(core_axis_name="c", subcore_axis_name="s")
_sc_params = pltpu.CompilerParams(use_tc_tiling_on_sc=False)


_M2 = 2 * N_EDGES
_GPW = _M2 // NW
_GIT = _GPW // CH


@functools.partial(
    pl.kernel,
    out_type=jax.ShapeDtypeStruct((_M2, D), jnp.float32),
    mesh=_sc_mesh,
    compiler_params=_sc_params,
    scratch_types=[
        pltpu.VMEM((CH,), jnp.int32),
        pltpu.VMEM((CH, D), jnp.float32),
        pltpu.SemaphoreType.DMA,
    ],
)
def _sc_gather(tab, idx, out, idx_v, rows_v, sem):
    c = lax.axis_index("c")
    s = lax.axis_index("s")
    wid = s * NC + c
    base = wid * _GPW

    def body(j, carry):
        off = base + j * CH
        pltpu.sync_copy(idx.at[pl.ds(off, CH)], idx_v)
        pltpu.async_copy(tab.at[idx_v], rows_v, sem).wait()
        pltpu.sync_copy(rows_v, out.at[pl.ds(off, CH)])
        return carry

    lax.fori_loop(0, _GIT, body, 0)


_SPW = N_EDGES // NW
_SIT = _SPW // CH
_RPT = NP // NS


@functools.partial(
    pl.kernel,
    out_type=jax.ShapeDtypeStruct((NC, NP, D), jnp.float32),
    mesh=_sc_mesh,
    compiler_params=_sc_params,
    scratch_types=[
        pltpu.VMEM((CH,), jnp.int32),
        pltpu.VMEM((CH, D), jnp.float32),
        pltpu.VMEM_SHARED((NP, D), jnp.float32),
        pltpu.SemaphoreType.DMA,
    ],
)
def _sc_scatter(msgs, dst, zeros, out, idx_v, rows_v, acc, sem):
    c = lax.axis_index("c")
    s = lax.axis_index("s")
    pltpu.sync_copy(zeros.at[pl.ds(s * _RPT, _RPT)], acc.at[pl.ds(s * _RPT, _RPT)])
    plsc.subcore_barrier()

    wid = s * NC + c
    base = wid * _SPW

    def body(j, carry):
        off = base + j * CH
        pltpu.sync_copy(dst.at[pl.ds(off, CH)], idx_v)
        pltpu.sync_copy(msgs.at[pl.ds(off, CH)], rows_v)
        pltpu.sync_copy(rows_v, acc.at[idx_v], add=True)
        return carry

    lax.fori_loop(0, _SIT, body, 0)
    plsc.subcore_barrier()
    pltpu.sync_copy(acc.at[pl.ds(s * _RPT, _RPT)], out.at[c, pl.ds(s * _RPT, _RPT)])


_RE = 8000


def _edge_body(g_ref, e_ref, wg_ref, we_ref, b1_ref, w2_ref, b2_ref, out_ref):
    h = jnp.dot(g_ref[...], wg_ref[...], preferred_element_type=jnp.float32)
    h = h + jnp.dot(e_ref[...], we_ref[...], preferred_element_type=jnp.float32)
    h = jnp.maximum(h + b1_ref[...], 0.0)
    out_ref[...] = (
        jnp.dot(h, w2_ref[...], preferred_element_type=jnp.float32) + b2_ref[...]
    )


def _edge_mlp(g16, E, wg, we, b1, w2, b2):
    grid = (N_EDGES // _RE,)
    return pl.pallas_call(
        _edge_body,
        grid=grid,
        in_specs=[
            pl.BlockSpec((_RE, 2 * D), lambda i: (i, 0)),
            pl.BlockSpec((_RE, 6), lambda i: (i, 0)),
            pl.BlockSpec((2 * D, 64), lambda i: (0, 0)),
            pl.BlockSpec((6, 64), lambda i: (0, 0)),
            pl.BlockSpec((1, 64), lambda i: (0, 0)),
            pl.BlockSpec((64, D), lambda i: (0, 0)),
            pl.BlockSpec((1, D), lambda i: (0, 0)),
        ],
        out_specs=pl.BlockSpec((_RE, D), lambda i: (i, 0)),
        out_shape=jax.ShapeDtypeStruct((N_EDGES, D), jnp.float32),
    )(g16, E, wg, we, b1, w2, b2)


_RV = NP // 16


def _vertex_body(vp_ref, agg_ref, w1a_ref, w1b_ref, b1_ref, w2_ref, b2_ref,
                 out_ref, sum_ref):
    i = pl.program_id(0)
    x = vp_ref[...]
    a = agg_ref[0] + agg_ref[1]
    h = jnp.dot(x, w1a_ref[...], preferred_element_type=jnp.float32)
    h = h + jnp.dot(a, w1b_ref[...], preferred_element_type=jnp.float32)
    h = jnp.maximum(h + b1_ref[...], 0.0)
    vn = jnp.dot(h, w2_ref[...], preferred_element_type=jnp.float32) + b2_ref[...]
    row = i * _RV + lax.broadcasted_iota(jnp.int32, (_RV, D), 0)
    vn = jnp.where(row < N_NODES, vn, 0.0)
    out_ref[...] = vn

    @pl.when(i == 0)
    def _():
        sum_ref[...] = jnp.zeros_like(sum_ref)

    sum_ref[...] += jnp.sum(vn, axis=0, keepdims=True)


def _vertex_mlp(vp, agg2, w1a, w1b, b1, w2, b2):
    grid = (NP // _RV,)
    return pl.pallas_call(
        _vertex_body,
        grid=grid,
        in_specs=[
            pl.BlockSpec((_RV, D), lambda i: (i, 0)),
            pl.BlockSpec((NC, _RV, D), lambda i: (0, i, 0)),
            pl.BlockSpec((D, 32), lambda i: (0, 0)),
            pl.BlockSpec((D, 32), lambda i: (0, 0)),
            pl.BlockSpec((1, 32), lambda i: (0, 0)),
            pl.BlockSpec((32, D), lambda i: (0, 0)),
            pl.BlockSpec((1, D), lambda i: (0, 0)),
        ],
        out_specs=[
            pl.BlockSpec((_RV, D), lambda i: (i, 0)),
            pl.BlockSpec((1, D), lambda i: (0, 0)),
        ],
        out_shape=[
            jax.ShapeDtypeStruct((NP, D), jnp.float32),
            jax.ShapeDtypeStruct((1, D), jnp.float32),
        ],
    )(vp, agg2, w1a, w1b, b1, w2, b2)


def _global_body(sums_ref, w1_ref, b1_ref, w2_ref, b2_ref, out_ref):
    m = jnp.sum(sums_ref[...], axis=0, keepdims=True) * (1.0 / N_NODES)
    h = jnp.dot(m, w1_ref[...], preferred_element_type=jnp.float32)
    h = jnp.maximum(h + b1_ref[...], 0.0)
    u = jnp.dot(h, w2_ref[...], preferred_element_type=jnp.float32) + b2_ref[...]
    out_ref[...] = jax.nn.sigmoid(u)


def _global_mlp(sums, w1, b1, w2, b2):
    return pl.pallas_call(
        _global_body,
        out_shape=jax.ShapeDtypeStruct((1, 1), jnp.float32),
    )(sums, w1, b1, w2, b2)


def kernel(V, E, edge_index, ew1, eb1, ew2, eb2, vw1, vb1, vw2, vb2,
           ow1, ob1, ow2, ob2):
    f32 = jnp.float32
    wg = jnp.zeros((2 * D, 64), f32)
    wg = wg.at[0:6].set(ew1[0:6]).at[D:D + 6].set(ew1[6:12])
    we = ew1[12:18]
    eb1r = eb1.reshape(1, 64)
    w2p = jnp.zeros((64, D), f32).at[:, 0:6].set(ew2)
    eb2p = jnp.zeros((1, D), f32).at[0, 0:6].set(eb2)

    w1a = jnp.zeros((D, 32), f32).at[0:6].set(vw1[0:6])
    w1b = jnp.zeros((D, 32), f32).at[0:6].set(vw1[6:12])
    vb1r = vb1.reshape(1, 32)
    vw2p = jnp.zeros((32, D), f32).at[:, 0:6].set(vw2)
    vb2p = jnp.zeros((1, D), f32).at[0, 0:6].set(vb2)

    ow1p = jnp.zeros((D, 32), f32).at[0:6].set(ow1)
    ob1r = ob1.reshape(1, 32)
    ob2r = ob2.reshape(1, 1)

    vp = jnp.zeros((NP, D), f32).at[0:N_NODES, 0:6].set(V)
    idx2 = edge_index.astype(jnp.int32).T.reshape(_M2)
    dst = edge_index[1].astype(jnp.int32)
    zeros = jnp.zeros((NP, D), f32)

    sums = None
    for _ in range(3):
        g = _sc_gather(vp, idx2)
        g16 = g.reshape(N_EDGES, 2 * D)
        msgs = _edge_mlp(g16, E, wg, we, eb1r, w2p, eb2p)
        agg2 = _sc_scatter(msgs, dst, zeros)
        vp, sums = _vertex_mlp(vp, agg2, w1a, w1b, vb1r, vw2p, vb2p)

    return _global_mlp(sums, ow1p, ob1r, ow2, ob2r)
